# baseline (speedup 1.0000x reference)
import jax
import jax.numpy as jnp
from jax.experimental import pallas as pl
from jax.experimental.pallas import tpu as pltpu
from jax.experimental.pallas import tpu_sc as plsc

_B = 4096
_L = 200
_D = 192
_DP = 256
_NT = _L * _D
_NC = 1
_LC = _L // _NC
_NCK = _B * _LC
_W = 128
_NBUF = 2
_NSUB = 32
_WPS = _NCK // (_W * _NSUB)
_IPS = _NCK // _NSUB
_GROUPS = _WPS // _NBUF


def _sc_body(tbl_hbm, idx_hbm, out_hbm, idx_ref, s0, s1, h0, h1, si, gs, ws):
    ss = [s0, s1]
    h64s = [h0, h1]
    core = jax.lax.axis_index("core")
    sub = jax.lax.axis_index("subcore")
    sid = core * 16 + sub
    wbase = sid * _WPS

    pltpu.async_copy(idx_hbm.at[pl.ds(sid * _IPS, _IPS)], idx_ref, si).wait()

    def start_gather(w, b):
        iv = idx_ref.at[pl.ds(w * _W, _W)]
        pltpu.async_copy(tbl_hbm.at[iv], ss[b], gs.at[b])

    def wait_gather(b):
        iv = idx_ref.at[pl.ds(0, _W)]
        pltpu.make_async_copy(tbl_hbm.at[iv], ss[b], gs.at[b]).wait()

    def repack(b):
        @pl.loop(0, _W)
        def _(r):
            for j in range(4):
                src = (pl.ds(r, 1), pl.ds(128 + j * 16, 16))
                dst = (pl.ds(r, 1), pl.ds(j * 16, 16))
                h64s[b].at[dst][...] = ss[b].at[src][...]

    def start_writes(w, b):
        rows = pl.ds((wbase + w) * _W, _W)
        pltpu.async_copy(
            ss[b].at[:, pl.ds(0, 128)], out_hbm.at[rows, pl.ds(0, 128)], ws.at[b]
        )
        pltpu.async_copy(h64s[b], out_hbm.at[rows, pl.ds(128, 64)], ws.at[b])

    def wait_writes(b):
        rows = pl.ds(wbase * _W, _W)
        pltpu.make_async_copy(
            ss[b].at[:, pl.ds(0, 128)], out_hbm.at[rows, pl.ds(0, 128)], ws.at[b]
        ).wait()
        pltpu.make_async_copy(
            h64s[b], out_hbm.at[rows, pl.ds(128, 64)], ws.at[b]
        ).wait()

    for b in range(_NBUF):
        start_gather(b, b)

    @pl.loop(1, _GROUPS)
    def _(g):
        for b in range(_NBUF):
            wait_gather(b)
            repack(b)
            start_writes((g - 1) * _NBUF + b, b)
        for b in range(_NBUF):
            wait_writes(b)
            start_gather(g * _NBUF + b, b)

    for b in range(_NBUF):
        wait_gather(b)
        repack(b)
        start_writes((_GROUPS - 1) * _NBUF + b, b)
    for b in range(_NBUF):
        wait_writes(b)


def _sc_chunk(table, idx_c):
    gather = pl.kernel(
        _sc_body,
        out_type=jax.ShapeDtypeStruct((_NCK, _D), jnp.float32),
        mesh=plsc.VectorSubcoreMesh(
            core_axis_name="core", subcore_axis_name="subcore"
        ),
        scratch_types=[
            pltpu.VMEM((_IPS,), jnp.int32),
            pltpu.VMEM((_W, _DP), jnp.float32),
            pltpu.VMEM((_W, _DP), jnp.float32),
            pltpu.VMEM((_W, 64), jnp.float32),
            pltpu.VMEM((_W, 64), jnp.float32),
            pltpu.SemaphoreType.DMA,
            pltpu.SemaphoreType.DMA((_NBUF,)),
            pltpu.SemaphoreType.DMA((_NBUF,)),
        ],
    )
    return gather(table, idx_c)


def _tc_body_first(im_ref, out_ref):
    for j in range(8):
        out_ref[pl.ds(j * _D, _D), :] = im_ref[:, j, :].T


def _tc_body_next(im_ref, prev_ref, out_ref):
    del prev_ref
    for j in range(8):
        out_ref[pl.ds(j * _D, _D), :] = im_ref[:, j, :].T


def _tc_transpose(im3, prev, c):
    im_spec = pl.BlockSpec((128, 8, _D), lambda bb, ls: (bb, ls, 0))
    out_spec = pl.BlockSpec(
        (8 * _D, 128), lambda bb, ls, _c=c: (_c * (_LC // 8) + ls, bb)
    )
    cp = pltpu.CompilerParams(dimension_semantics=("parallel", "parallel"))
    if prev is None:
        return pl.pallas_call(
            _tc_body_first,
            out_shape=jax.ShapeDtypeStruct((_NT, _B), jnp.float32),
            grid=(_B // 128, _LC // 8),
            in_specs=[im_spec],
            out_specs=out_spec,
            compiler_params=cp,
        )(im3)
    return pl.pallas_call(
        _tc_body_next,
        out_shape=jax.ShapeDtypeStruct((_NT, _B), jnp.float32),
        grid=(_B // 128, _LC // 8),
        in_specs=[im_spec, pl.BlockSpec(memory_space=pl.ANY)],
        out_specs=out_spec,
        input_output_aliases={1: 0},
        compiler_params=cp,
    )(im3, prev)


def kernel(x, emb1_weight, emb2_weight):
    table = jnp.concatenate(
        (
            emb1_weight,
            emb2_weight,
            jnp.zeros((emb1_weight.shape[0], _DP - _D), emb1_weight.dtype),
        ),
        axis=1,
    )
    xi = x.astype(jnp.int32)

    out_t = None
    for c in range(_NC):
        idx_c = xi[:, c * _LC : (c + 1) * _LC].reshape(_NCK)
        im = _sc_chunk(table, idx_c)
        im3 = im.reshape(_B, _LC, _D)
        out_t = _tc_transpose(im3, out_t, c)

    return jnp.transpose(out_t.reshape(1, _L, _D, _B), (3, 0, 1, 2))


# speedup vs baseline: 1.2422x; 1.2422x over previous
import jax
import jax.numpy as jnp
from jax.experimental import pallas as pl
from jax.experimental.pallas import tpu as pltpu
from jax.experimental.pallas import tpu_sc as plsc

_B = 4096
_L = 200
_D = 192
_DP = 256
_N = _B * _L
_W = 128
_NBUF = 2
_NSUB = 32
_WPS = _N // (_W * _NSUB)
_IPS = _N // _NSUB
_GROUPS = _WPS // _NBUF


def _gather_body(tbl_hbm, idx_hbm, out_hbm, idx_ref, s0, s1, h0, h1, si, gs, ws):
    ss = [s0, s1]
    h64s = [h0, h1]
    core = jax.lax.axis_index("core")
    sub = jax.lax.axis_index("subcore")
    sid = core * 16 + sub
    wbase = sid * _WPS

    pltpu.async_copy(idx_hbm.at[pl.ds(sid * _IPS, _IPS)], idx_ref, si).wait()

    def start_gather(w, b):
        iv = idx_ref.at[pl.ds(w * _W, _W)]
        pltpu.async_copy(tbl_hbm.at[iv], ss[b], gs.at[b])

    def wait_gather(b):
        iv = idx_ref.at[pl.ds(0, _W)]
        pltpu.make_async_copy(tbl_hbm.at[iv], ss[b], gs.at[b]).wait()

    def repack(b):
        @pl.loop(0, _W)
        def _(r):
            for j in range(4):
                src = (pl.ds(r, 1), pl.ds(128 + j * 16, 16))
                dst = (pl.ds(r, 1), pl.ds(j * 16, 16))
                h64s[b].at[dst][...] = ss[b].at[src][...]

    def start_writes(w, b):
        rows = pl.ds((wbase + w) * _W, _W)
        pltpu.async_copy(
            ss[b].at[:, pl.ds(0, 128)], out_hbm.at[rows, pl.ds(0, 128)], ws.at[b]
        )
        pltpu.async_copy(h64s[b], out_hbm.at[rows, pl.ds(128, 64)], ws.at[b])

    def wait_writes(b):
        rows = pl.ds(wbase * _W, _W)
        pltpu.make_async_copy(
            ss[b].at[:, pl.ds(0, 128)], out_hbm.at[rows, pl.ds(0, 128)], ws.at[b]
        ).wait()
        pltpu.make_async_copy(
            h64s[b], out_hbm.at[rows, pl.ds(128, 64)], ws.at[b]
        ).wait()

    for b in range(_NBUF):
        start_gather(b, b)

    @pl.loop(1, _GROUPS)
    def _(g):
        for b in range(_NBUF):
            wait_gather(b)
            repack(b)
            start_writes((g - 1) * _NBUF + b, b)
        for b in range(_NBUF):
            wait_writes(b)
            start_gather(g * _NBUF + b, b)

    for b in range(_NBUF):
        wait_gather(b)
        repack(b)
        start_writes((_GROUPS - 1) * _NBUF + b, b)
    for b in range(_NBUF):
        wait_writes(b)


def kernel(x, emb1_weight, emb2_weight):
    table = jnp.concatenate(
        (
            emb1_weight,
            emb2_weight,
            jnp.zeros((emb1_weight.shape[0], _DP - _D), emb1_weight.dtype),
        ),
        axis=1,
    )
    idx = x.reshape(_N).astype(jnp.int32)

    gather = pl.kernel(
        _gather_body,
        out_type=jax.ShapeDtypeStruct((_N, _D), jnp.float32),
        mesh=plsc.VectorSubcoreMesh(
            core_axis_name="core", subcore_axis_name="subcore"
        ),
        scratch_types=[
            pltpu.VMEM((_IPS,), jnp.int32),
            pltpu.VMEM((_W, _DP), jnp.float32),
            pltpu.VMEM((_W, _DP), jnp.float32),
            pltpu.VMEM((_W, 64), jnp.float32),
            pltpu.VMEM((_W, 64), jnp.float32),
            pltpu.SemaphoreType.DMA,
            pltpu.SemaphoreType.DMA((_NBUF,)),
            pltpu.SemaphoreType.DMA((_NBUF,)),
        ],
    )
    out = gather(table, idx)
    return out.reshape(_B, 1, _L, _D)
